# baseline (speedup 1.0000x reference)
import functools

import jax
import jax.numpy as jnp
from jax.experimental import pallas as pl


def _expert_matmul_kernel(x_ref, w_ref, b_ref, o_ref):
    w = w_ref[0]
    B = x_ref.shape[0]
    for bb in range(B):
        acc = jax.lax.dot_general(
            x_ref[bb, 0], w,
            dimension_numbers=(((1,), (1,)), ((), ())),
            preferred_element_type=jnp.float32,
        )
        o_ref[bb, 0] = acc + b_ref[0]


@jax.jit
def kernel(x, W, b):
    B, E, N, D = x.shape
    BF = 512
    F = D // BF

    b3 = b.reshape(E, 1, D)
    return pl.pallas_call(
        _expert_matmul_kernel,
        grid=(E, F),
        in_specs=[
            pl.BlockSpec((B, 1, N, D), lambda e, f: (0, e, 0, 0)),
            pl.BlockSpec((1, BF, D), lambda e, f: (e, f, 0)),
            pl.BlockSpec((1, 1, BF), lambda e, f: (e, 0, f)),
        ],
        out_specs=pl.BlockSpec((B, 1, N, BF), lambda e, f: (0, e, 0, f)),
        out_shape=jax.ShapeDtypeStruct((B, E, N, D), x.dtype),
    )(x, W, b3)


# speedup vs baseline: 1.0422x; 1.0422x over previous
import functools

import jax
import jax.numpy as jnp
from jax.experimental import pallas as pl


def _expert_matmul_kernel(x_ref, w_ref, b_ref, o_ref):
    w = w_ref[0]
    B = x_ref.shape[0]
    for bb in range(B):
        acc = jax.lax.dot_general(
            x_ref[bb, 0], w,
            dimension_numbers=(((1,), (1,)), ((), ())),
            preferred_element_type=jnp.float32,
        )
        o_ref[bb, 0] = acc + b_ref[0]


@jax.jit
def kernel(x, W, b):
    B, E, N, D = x.shape
    BF = 1024
    F = D // BF

    b3 = b.reshape(E, 1, D)
    return pl.pallas_call(
        _expert_matmul_kernel,
        grid=(E, F),
        in_specs=[
            pl.BlockSpec((B, 1, N, D), lambda e, f: (0, e, 0, 0)),
            pl.BlockSpec((1, BF, D), lambda e, f: (e, f, 0)),
            pl.BlockSpec((1, 1, BF), lambda e, f: (e, 0, f)),
        ],
        out_specs=pl.BlockSpec((B, 1, N, BF), lambda e, f: (0, e, 0, f)),
        out_shape=jax.ShapeDtypeStruct((B, E, N, D), x.dtype),
    )(x, W, b3)


# speedup vs baseline: 1.0428x; 1.0006x over previous
import functools

import jax
import jax.numpy as jnp
from jax.experimental import pallas as pl


def _expert_matmul_kernel(x0_ref, x1_ref, w0_ref, w1_ref, b_ref, o_ref):
    BF2 = w0_ref.shape[1]
    for bb, x_ref in enumerate((x0_ref, x1_ref)):
        x = x_ref[0, 0]
        for wi, w_ref in enumerate((w0_ref, w1_ref)):
            acc = jax.lax.dot_general(
                x, w_ref[0],
                dimension_numbers=(((1,), (1,)), ((), ())),
                preferred_element_type=jnp.float32,
            )
            o_ref[bb, 0, :, wi * BF2:(wi + 1) * BF2] = (
                acc + b_ref[0, :, wi * BF2:(wi + 1) * BF2])


@jax.jit
def kernel(x, W, b):
    B, E, N, D = x.shape
    BF = 1024
    BF2 = BF // 2
    F = D // BF

    b3 = b.reshape(E, 1, D)
    return pl.pallas_call(
        _expert_matmul_kernel,
        grid=(E, F),
        in_specs=[
            pl.BlockSpec((1, 1, N, D), lambda e, f: (0, e, 0, 0)),
            pl.BlockSpec((1, 1, N, D), lambda e, f: (1, e, 0, 0)),
            pl.BlockSpec((1, BF2, D), lambda e, f: (e, 2 * f, 0)),
            pl.BlockSpec((1, BF2, D), lambda e, f: (e, 2 * f + 1, 0)),
            pl.BlockSpec((1, 1, BF), lambda e, f: (e, 0, f)),
        ],
        out_specs=pl.BlockSpec((B, 1, N, BF), lambda e, f: (0, e, 0, f)),
        out_shape=jax.ShapeDtypeStruct((B, E, N, D), x.dtype),
    )(x, x, W, W, b3)


# speedup vs baseline: 1.2568x; 1.2053x over previous
import functools

import jax
import jax.numpy as jnp
from jax.experimental import pallas as pl
from jax.experimental.pallas import tpu as pltpu

_FC = 4
_NW = 4


def _experts_kernel(x_hbm, w_hbm, b_vmem, o_hbm,
                    xb, wb, ob, xsem, wsem, osem):
    B, E, N, D = x_hbm.shape
    BFC = D // _FC
    G = E * _FC

    def w_copy(g):
        e, fc = divmod(g, _FC)
        return pltpu.make_async_copy(
            w_hbm.at[e, pl.ds(fc * BFC, BFC), :], wb.at[g % _NW],
            wsem.at[g % _NW])

    def x_copy(e):
        return pltpu.make_async_copy(
            x_hbm.at[:, e], xb.at[e % 2], xsem.at[e % 2])

    def o_copy(g):
        e, fc = divmod(g, _FC)
        return pltpu.make_async_copy(
            ob.at[g % 2], o_hbm.at[:, e, :, pl.ds(fc * BFC, BFC)],
            osem.at[g % 2])

    x_copy(0).start()
    for g in range(min(_NW, G)):
        w_copy(g).start()

    for g in range(G):
        e, fc = divmod(g, _FC)
        if fc == 0:
            x_copy(e).wait()
        w_copy(g).wait()
        if g >= 2:
            o_copy(g - 2).wait()
        bias_row = b_vmem[e, 0, fc * BFC:(fc + 1) * BFC]
        for bb in range(B):
            acc = jax.lax.dot_general(
                xb[e % 2, bb], wb[g % _NW],
                dimension_numbers=(((1,), (1,)), ((), ())),
                preferred_element_type=jnp.float32,
            )
            ob[g % 2, bb] = acc + bias_row[None, :]
        o_copy(g).start()
        if g + _NW < G:
            w_copy(g + _NW).start()
        if fc == 0 and e + 1 < E:
            x_copy(e + 1).start()

    o_copy(G - 2).wait()
    o_copy(G - 1).wait()


@jax.jit
def kernel(x, W, b):
    B, E, N, D = x.shape
    BFC = D // _FC
    b3 = b.reshape(E, 1, D)
    return pl.pallas_call(
        _experts_kernel,
        in_specs=[
            pl.BlockSpec(memory_space=pltpu.HBM),
            pl.BlockSpec(memory_space=pltpu.HBM),
            pl.BlockSpec(memory_space=pltpu.VMEM),
        ],
        out_specs=pl.BlockSpec(memory_space=pltpu.HBM),
        out_shape=jax.ShapeDtypeStruct((B, E, N, D), x.dtype),
        scratch_shapes=[
            pltpu.VMEM((2, B, N, D), jnp.float32),
            pltpu.VMEM((_NW, BFC, D), jnp.float32),
            pltpu.VMEM((2, B, N, BFC), jnp.float32),
            pltpu.SemaphoreType.DMA((2,)),
            pltpu.SemaphoreType.DMA((_NW,)),
            pltpu.SemaphoreType.DMA((2,)),
        ],
        compiler_params=pltpu.CompilerParams(
            vmem_limit_bytes=100 * 1024 * 1024),
    )(x, W, b3)
